# baseline (speedup 1.0000x reference)
import functools

import jax
import jax.numpy as jnp
from jax import lax
from jax.experimental import pallas as pl
from jax.experimental.pallas import tpu as pltpu
from jax.experimental.pallas import tpu_sc as plsc

NU = 10000
NI = 10000
D = 64
NE = 320000
RANK = 5
RPAD = 16
B = 1024
K = 5
BF = B * K
GB = B + BF
L2_REG = 1e-4

NCORE = 2
NSUB = 16
SUB = 128
EPT = 20480
NE_PAD = EPT * NSUB
ROWS_PT = EPT // SUB
CR = 2
CHUNK = CR * SUB
NCH = ROWS_PT // CR
GPT = GB // NSUB
GROWS = GPT // SUB
ZPT = NU // NSUB
TAB_ROWS = 10112
ACC_ROWS = TAB_ROWS

@functools.lru_cache(maxsize=1)
def _get_sc_kernel():
    mesh = plsc.VectorSubcoreMesh(
        core_axis_name="c", subcore_axis_name="s",
        num_cores=NCORE, num_subcores=NSUB,
    )
    return pl.kernel(
        _sc_segment_and_gather,
        out_type=(
            jax.ShapeDtypeStruct((NCORE, GB, D), jnp.float32),
            jax.ShapeDtypeStruct((NCORE, GB, RPAD), jnp.float32),
        ),
        mesh=mesh,
        scratch_types=[
            pltpu.VMEM_SHARED((ACC_ROWS, D), jnp.float32),
            pltpu.VMEM_SHARED((TAB_ROWS, D), jnp.float32),
            pltpu.VMEM((CR, SUB), jnp.int32),
            pltpu.VMEM((CR, SUB), jnp.int32),
            pltpu.VMEM((CR, SUB), jnp.int32),
            pltpu.VMEM((CR, SUB), jnp.int32),
            pltpu.VMEM((CHUNK, D), jnp.float32),
            pltpu.VMEM((CHUNK, D), jnp.float32),
            pltpu.VMEM((GROWS, SUB), jnp.int32),
            pltpu.VMEM((GPT, RPAD), jnp.float32),
            pltpu.SemaphoreType.DMA,
            pltpu.SemaphoreType.DMA,
            pltpu.SemaphoreType.DMA,
            pltpu.SemaphoreType.DMA,
            pltpu.SemaphoreType.DMA,
            pltpu.SemaphoreType.DMA,
        ],
        compiler_params=pltpu.CompilerParams(use_tc_tiling_on_sc=False),
    )


def _sc_segment_and_gather(
    item_tab, user_tab, row_idx, col_idx, bidx, svd_u, svd_v, zrows,
    emb_out, svd_out,
    acc, tab_s, gid_a, gid_b, sid_a, sid_b, rows_a, rows_b,
    bi2, sr_v,
    sem_ia, sem_ib, sem_ga, sem_gb, sem_sa, sem_sb,
):
    cid = lax.axis_index("c")
    sid = lax.axis_index("s")

    def run(core, tab_hbm, g_hbm, s_hbm, svd_hbm):
        dz = pltpu.async_copy(zrows, acc.at[pl.ds(sid * ZPT, ZPT)], sem_sa)
        dt = pltpu.async_copy(
            tab_hbm.at[pl.ds(sid * ZPT, ZPT)], tab_s.at[pl.ds(sid * ZPT, ZPT)],
            sem_sb,
        )
        dz.wait()
        dt.wait()
        plsc.subcore_barrier()

        grp0 = sid * NCH

        def fire_idx(c, gv, sv, sem_i):
            pltpu.async_copy(g_hbm.at[grp0 + c], gv, sem_i)
            pltpu.async_copy(s_hbm.at[grp0 + c], sv, sem_i)

        def fire_gather(gv, sv, rowsv, sem_i, sem_g):
            pltpu.make_async_copy(g_hbm.at[0], gv, sem_i).wait()
            pltpu.make_async_copy(g_hbm.at[0], sv, sem_i).wait()
            for j in range(CR):
                pltpu.async_copy(
                    tab_s.at[gv.at[j]], rowsv.at[pl.ds(j * SUB, SUB)], sem_g
                )

        def fire_scatter(sv, rowsv, sem_g, sem_s):
            pltpu.make_async_copy(item_tab.at[pl.ds(0, CHUNK)], rowsv, sem_g).wait()
            for j in range(CR):
                pltpu.async_copy(
                    rowsv.at[pl.ds(j * SUB, SUB)], acc.at[sv.at[j]], sem_s,
                    add=True,
                )

        def wait_scatter(rowsv, sem_s):
            pltpu.make_async_copy(item_tab.at[pl.ds(0, CHUNK)], rowsv, sem_s).wait()

        fire_idx(0, gid_a, sid_a, sem_ia)
        fire_gather(gid_a, sid_a, rows_a, sem_ia, sem_ga)
        fire_idx(1, gid_b, sid_b, sem_ib)

        def pair(h, carry):
            c = 2 * h
            fire_gather(gid_b, sid_b, rows_b, sem_ib, sem_gb)
            fire_scatter(sid_a, rows_a, sem_ga, sem_sa)
            wait_scatter(rows_a, sem_sa)
            fire_idx(c + 2, gid_a, sid_a, sem_ia)
            fire_gather(gid_a, sid_a, rows_a, sem_ia, sem_ga)
            fire_scatter(sid_b, rows_b, sem_gb, sem_sb)
            wait_scatter(rows_b, sem_sb)
            fire_idx(c + 3, gid_b, sid_b, sem_ib)
            return carry

        lax.fori_loop(0, NCH // 2 - 1, pair, 0)
        fire_gather(gid_b, sid_b, rows_b, sem_ib, sem_gb)
        fire_scatter(sid_a, rows_a, sem_ga, sem_sa)
        wait_scatter(rows_a, sem_sa)
        fire_scatter(sid_b, rows_b, sem_gb, sem_sb)
        wait_scatter(rows_b, sem_sb)

        plsc.subcore_barrier()

        pltpu.sync_copy(bidx.at[core, sid], bi2)
        svd_descs = [
            pltpu.async_copy(
                svd_hbm.at[bi2.at[j]], sr_v.at[pl.ds(j * SUB, SUB)], sem_gb
            )
            for j in range(GROWS)
        ]
        da = pltpu.async_copy(acc.at[bi2.at[0]], rows_a.at[pl.ds(0, SUB)], sem_ia)
        db = pltpu.async_copy(acc.at[bi2.at[1]], rows_a.at[pl.ds(SUB, SUB)], sem_ia)
        dc = pltpu.async_copy(acc.at[bi2.at[CR]], rows_b.at[pl.ds(0, SUB)], sem_ib)
        da.wait()
        db.wait()
        dc.wait()
        d1 = pltpu.async_copy(
            rows_a, emb_out.at[core, pl.ds(sid * GPT, CHUNK)], sem_sa
        )
        d2 = pltpu.async_copy(
            rows_b.at[pl.ds(0, SUB)],
            emb_out.at[core, pl.ds(sid * GPT + CHUNK, SUB)],
            sem_sb,
        )
        for d in svd_descs:
            d.wait()
        d3 = pltpu.async_copy(sr_v, svd_out.at[core, pl.ds(sid * GPT, GPT)], sem_ga)
        d1.wait()
        d2.wait()
        d3.wait()

    @pl.when(cid == 0)
    def _zu():
        run(0, item_tab, col_idx, row_idx, svd_u)

    @pl.when(cid == 1)
    def _zi():
        run(1, user_tab, row_idx, col_idx, svd_v)


def _nrm(x):
    n = jnp.sqrt(jnp.sum(x * x, axis=1, keepdims=True))
    return x / jnp.maximum(n, 1e-12)


def _tc1_prep(
    emb_ref, svd_ref, uT, vT, utab, itab, lab,
    scores_ref, rec_ref, embl_ref, tot_ref, gnn_scr, hyp_scr,
):
    zu_b = emb_ref[0, :B, :]
    zu3 = emb_ref[0, B:, :].reshape(B, K, D)
    zi = emb_ref[1, :BF, :]
    zi3 = zi.reshape(B, K, D)
    usvd_b = svd_ref[0, :B, :]
    vsvd_b = svd_ref[1, :BF, :]
    P_u = lax.dot_general(
        uT[...], utab[...], (((0,), (0,)), ((), ())),
        preferred_element_type=jnp.float32,
    )
    P_i = lax.dot_general(
        vT[...], itab[...], (((0,), (0,)), ((), ())),
        preferred_element_type=jnp.float32,
    )
    gnn_u = _nrm(jnp.dot(usvd_b, P_i, preferred_element_type=jnp.float32))
    hyp_u = _nrm(zu_b)
    gnn_scr[...] = _nrm(jnp.dot(vsvd_b, P_u, preferred_element_type=jnp.float32))
    hyp_scr[...] = _nrm(zi)

    pos_u = jnp.exp(jnp.sum(gnn_u * hyp_u, axis=1))
    neg_u = jnp.sum(
        jnp.exp(
            lax.dot_general(
                gnn_u, hyp_u, (((1,), (1,)), ((), ())),
                preferred_element_type=jnp.float32,
            )
        ),
        axis=1,
    )
    loss_u = jnp.mean(-jnp.log(pos_u / (neg_u + 1e-8) + 1e-8))

    scores = jnp.sum(zu3 * zi3, axis=2)
    sm = scores - jnp.max(scores, axis=1, keepdims=True)
    es = jnp.exp(sm)
    probs = es / jnp.sum(es, axis=1, keepdims=True)
    pm = jnp.max(probs, axis=1, keepdims=True)
    lse = pm + jnp.log(jnp.sum(jnp.exp(probs - pm), axis=1, keepdims=True))
    logp = probs - lse

    labv = lab[...]
    lm = jnp.max(labv, axis=1, keepdims=True)
    idxs = lax.broadcasted_iota(jnp.int32, (B, K), 1)
    cand = jnp.where(labv >= lm, idxs, K)
    tgt = jnp.min(cand, axis=1, keepdims=True)
    onehot = (idxs == tgt).astype(jnp.float32)
    rec = -jnp.mean(jnp.sum(logp * onehot, axis=1))

    reg = (jnp.sum(zu_b ** 2) + jnp.sum(zi ** 2)) * 0.5
    embl = reg * (L2_REG / B)

    scores_ref[...] = scores
    rec_ref[...] = jnp.full((1, 1), rec)
    embl_ref[...] = jnp.full((1, 1), embl)
    tot_ref[...] = jnp.full((1, 1), rec + embl + 0.5 * loss_u)


_BLK = 512
_NBLK = BF // _BLK


def _tc_body(
    emb_ref, svd_ref, uT, vT, utab, itab, lab,
    scores_ref, rec_ref, embl_ref, tot_ref, gnn_scr, hyp_scr,
):
    i = pl.program_id(0)

    @pl.when(i == 0)
    def _prep():
        _tc1_prep(
            emb_ref, svd_ref, uT, vT, utab, itab, lab,
            scores_ref, rec_ref, embl_ref, tot_ref, gnn_scr, hyp_scr,
        )

    @pl.when(i > 0)
    def _neg_i_block():
        g = gnn_scr[pl.ds((i - 1) * _BLK, _BLK), :]
        hr = hyp_scr[pl.ds((i - 1) * _BLK, _BLK), :]
        pos = jnp.exp(jnp.sum(g * hr, axis=1))
        neg = jnp.sum(
            jnp.exp(
                lax.dot_general(
                    g, hyp_scr[...], (((1,), (1,)), ((), ())),
                    preferred_element_type=jnp.float32,
                )
            ),
            axis=1,
        )
        s = jnp.sum(-jnp.log(pos / (neg + 1e-8) + 1e-8))
        tot_ref[...] = tot_ref[...] + jnp.full((1, 1), 0.5 * s / BF)


def kernel(user_table, item_table, u_svd, v_svd, users, items, label, ui_row, ui_col):
    users = users.astype(jnp.int32)
    items_flat = items.reshape(-1).astype(jnp.int32)
    ui_row = ui_row.astype(jnp.int32)
    ui_col = ui_col.astype(jnp.int32)

    u_svd_p = jnp.pad(u_svd, ((0, 0), (0, RPAD - RANK)))
    v_svd_p = jnp.pad(v_svd, ((0, 0), (0, RPAD - RANK)))

    row_idx = jnp.pad(ui_row, (0, NE_PAD - NE), constant_values=NU).reshape(
        NE_PAD // CHUNK, CR, SUB
    )
    col_idx = jnp.pad(ui_col, (0, NE_PAD - NE), constant_values=NU).reshape(
        NE_PAD // CHUNK, CR, SUB
    )

    users_rep = jnp.repeat(users, K)
    bidx0 = jnp.concatenate([users, users_rep])
    bidx1 = jnp.concatenate([items_flat, items_flat[:B]])
    bidx = jnp.stack([bidx0, bidx1]).reshape(NCORE, NSUB, GROWS, SUB)

    zrows = jnp.zeros((ZPT, D), jnp.float32)

    emb_b, svd_b = _get_sc_kernel()(
        item_table, user_table, row_idx, col_idx, bidx, u_svd_p, v_svd_p, zrows
    )

    def full(shape):
        return pl.BlockSpec(shape, lambda i, _n=len(shape): (0,) * _n)

    scores, rec, embl, tot = pl.pallas_call(
        _tc_body,
        grid=(_NBLK + 1,),
        in_specs=[
            full((NCORE, GB, D)), full((NCORE, GB, RPAD)),
            full((NU, RPAD)), full((NI, RPAD)),
            full((NU, D)), full((NI, D)), full((B, K)),
        ],
        out_specs=[full((B, K)), full((1, 1)), full((1, 1)), full((1, 1))],
        out_shape=(
            jax.ShapeDtypeStruct((B, K), jnp.float32),
            jax.ShapeDtypeStruct((1, 1), jnp.float32),
            jax.ShapeDtypeStruct((1, 1), jnp.float32),
            jax.ShapeDtypeStruct((1, 1), jnp.float32),
        ),
        scratch_shapes=[
            pltpu.VMEM((BF, D), jnp.float32),
            pltpu.VMEM((BF, D), jnp.float32),
        ],
    )(emb_b, svd_b, u_svd_p, v_svd_p, user_table, item_table, label)

    return (tot[0, 0], scores, rec[0, 0], embl[0, 0])


# speedup vs baseline: 1.0256x; 1.0256x over previous
import functools

import jax
import jax.numpy as jnp
from jax import lax
from jax.experimental import pallas as pl
from jax.experimental.pallas import tpu as pltpu
from jax.experimental.pallas import tpu_sc as plsc

NU = 10000
NI = 10000
D = 64
NE = 320000
RANK = 5
RPAD = 16
B = 1024
K = 5
BF = B * K
GB = B + BF
L2_REG = 1e-4

NCORE = 2
NSUB = 16
SUB = 128
EPT = 20480
NE_PAD = EPT * NSUB
ROWS_PT = EPT // SUB
CR = 2
CHUNK = CR * SUB
NCH = ROWS_PT // CR
GPT = GB // NSUB
GROWS = GPT // SUB
ZPT = NU // NSUB
TAB_ROWS = 10112
ACC_ROWS = TAB_ROWS

@functools.lru_cache(maxsize=1)
def _get_sc_kernel():
    mesh = plsc.VectorSubcoreMesh(
        core_axis_name="c", subcore_axis_name="s",
        num_cores=NCORE, num_subcores=NSUB,
    )
    return pl.kernel(
        _sc_segment_and_gather,
        out_type=(
            jax.ShapeDtypeStruct((NCORE, GB, D), jnp.float32),
            jax.ShapeDtypeStruct((NCORE, GB, RPAD), jnp.float32),
        ),
        mesh=mesh,
        scratch_types=[
            pltpu.VMEM_SHARED((ACC_ROWS, D), jnp.float32),
            pltpu.VMEM_SHARED((TAB_ROWS, D), jnp.float32),
            pltpu.VMEM((CR, SUB), jnp.int32),
            pltpu.VMEM((CR, SUB), jnp.int32),
            pltpu.VMEM((CR, SUB), jnp.int32),
            pltpu.VMEM((CR, SUB), jnp.int32),
            pltpu.VMEM((CHUNK, D), jnp.float32),
            pltpu.VMEM((CHUNK, D), jnp.float32),
            pltpu.VMEM((GROWS, SUB), jnp.int32),
            pltpu.VMEM((GPT, RPAD), jnp.float32),
            pltpu.SemaphoreType.DMA,
            pltpu.SemaphoreType.DMA,
            pltpu.SemaphoreType.DMA,
            pltpu.SemaphoreType.DMA,
            pltpu.SemaphoreType.DMA,
            pltpu.SemaphoreType.DMA,
        ],
        compiler_params=pltpu.CompilerParams(use_tc_tiling_on_sc=False),
    )


def _sc_segment_and_gather(
    item_tab, user_tab, row_idx, col_idx, bidx, svd_u, svd_v, zrows,
    emb_out, svd_out,
    acc, tab_s, gid_a, gid_b, sid_a, sid_b, rows_a, rows_b,
    bi2, sr_v,
    sem_ia, sem_ib, sem_ga, sem_gb, sem_sa, sem_sb,
):
    cid = lax.axis_index("c")
    sid = lax.axis_index("s")

    def run(core, tab_hbm, g_hbm, s_hbm, svd_hbm):
        dz = pltpu.async_copy(zrows, acc.at[pl.ds(sid * ZPT, ZPT)], sem_sa)
        dt = pltpu.async_copy(
            tab_hbm.at[pl.ds(sid * ZPT, ZPT)], tab_s.at[pl.ds(sid * ZPT, ZPT)],
            sem_sb,
        )
        dz.wait()
        dt.wait()
        plsc.subcore_barrier()

        grp0 = sid * NCH

        def fire_idx(c, gv, sv, sem_i):
            pltpu.async_copy(g_hbm.at[grp0 + c], gv, sem_i)
            pltpu.async_copy(s_hbm.at[grp0 + c], sv, sem_i)

        def fire_gather(gv, sv, rowsv, sem_i, sem_g):
            pltpu.make_async_copy(g_hbm.at[0], gv, sem_i).wait()
            pltpu.make_async_copy(g_hbm.at[0], sv, sem_i).wait()
            for j in range(CR):
                pltpu.async_copy(
                    tab_s.at[gv.at[j]], rowsv.at[pl.ds(j * SUB, SUB)], sem_g
                )

        def fire_scatter(sv, rowsv, sem_g, sem_s):
            pltpu.make_async_copy(item_tab.at[pl.ds(0, CHUNK)], rowsv, sem_g).wait()
            for j in range(CR):
                pltpu.async_copy(
                    rowsv.at[pl.ds(j * SUB, SUB)], acc.at[sv.at[j]], sem_s,
                    add=True,
                )

        def wait_scatter(rowsv, sem_s):
            pltpu.make_async_copy(item_tab.at[pl.ds(0, CHUNK)], rowsv, sem_s).wait()

        fire_idx(0, gid_a, sid_a, sem_ia)
        fire_gather(gid_a, sid_a, rows_a, sem_ia, sem_ga)
        fire_idx(1, gid_b, sid_b, sem_ib)

        def pair(h, carry):
            c = 2 * h
            fire_gather(gid_b, sid_b, rows_b, sem_ib, sem_gb)
            fire_scatter(sid_a, rows_a, sem_ga, sem_sa)
            wait_scatter(rows_a, sem_sa)
            fire_idx(c + 2, gid_a, sid_a, sem_ia)
            fire_gather(gid_a, sid_a, rows_a, sem_ia, sem_ga)
            fire_scatter(sid_b, rows_b, sem_gb, sem_sb)
            wait_scatter(rows_b, sem_sb)
            fire_idx(c + 3, gid_b, sid_b, sem_ib)
            return carry

        lax.fori_loop(0, NCH // 2 - 1, pair, 0)
        fire_gather(gid_b, sid_b, rows_b, sem_ib, sem_gb)
        fire_scatter(sid_a, rows_a, sem_ga, sem_sa)
        wait_scatter(rows_a, sem_sa)
        fire_scatter(sid_b, rows_b, sem_gb, sem_sb)
        wait_scatter(rows_b, sem_sb)

        plsc.subcore_barrier()

        pltpu.sync_copy(bidx.at[core, sid], bi2)
        svd_descs = [
            pltpu.async_copy(
                svd_hbm.at[bi2.at[j]], sr_v.at[pl.ds(j * SUB, SUB)], sem_gb
            )
            for j in range(GROWS)
        ]
        da = pltpu.async_copy(acc.at[bi2.at[0]], rows_a.at[pl.ds(0, SUB)], sem_ia)
        db = pltpu.async_copy(acc.at[bi2.at[1]], rows_a.at[pl.ds(SUB, SUB)], sem_ia)
        dc = pltpu.async_copy(acc.at[bi2.at[CR]], rows_b.at[pl.ds(0, SUB)], sem_ib)
        da.wait()
        db.wait()
        dc.wait()
        d1 = pltpu.async_copy(
            rows_a, emb_out.at[core, pl.ds(sid * GPT, CHUNK)], sem_sa
        )
        d2 = pltpu.async_copy(
            rows_b.at[pl.ds(0, SUB)],
            emb_out.at[core, pl.ds(sid * GPT + CHUNK, SUB)],
            sem_sb,
        )
        for d in svd_descs:
            d.wait()
        d3 = pltpu.async_copy(sr_v, svd_out.at[core, pl.ds(sid * GPT, GPT)], sem_ga)
        d1.wait()
        d2.wait()
        d3.wait()

    @pl.when(cid == 0)
    def _zu():
        run(0, item_tab, col_idx, row_idx, svd_u)

    @pl.when(cid == 1)
    def _zi():
        run(1, user_tab, row_idx, col_idx, svd_v)


def _nrm(x):
    n = jnp.sqrt(jnp.sum(x * x, axis=1, keepdims=True))
    return x / jnp.maximum(n, 1e-12)


def _tc1_prep(
    emb_ref, svd_ref, uT, vT, utab, itab, lab,
    scores_ref, rec_ref, embl_ref, tot_ref, gnn_scr, hyp_scr,
):
    zu_b = emb_ref[0, :B, :]
    zu3 = emb_ref[0, B:, :].reshape(B, K, D)
    zi = emb_ref[1, :BF, :]
    zi3 = zi.reshape(B, K, D)
    usvd_b = svd_ref[0, :B, :]
    vsvd_b = svd_ref[1, :BF, :]
    P_u = jnp.dot(uT[...], utab[...], preferred_element_type=jnp.float32)
    P_i = jnp.dot(vT[...], itab[...], preferred_element_type=jnp.float32)
    gnn_u = _nrm(jnp.dot(usvd_b, P_i, preferred_element_type=jnp.float32))
    hyp_u = _nrm(zu_b)
    gnn_scr[...] = _nrm(jnp.dot(vsvd_b, P_u, preferred_element_type=jnp.float32))
    hyp_scr[...] = _nrm(zi)

    pos_u = jnp.exp(jnp.sum(gnn_u * hyp_u, axis=1))
    neg_u = jnp.sum(
        jnp.exp(
            lax.dot_general(
                gnn_u, hyp_u, (((1,), (1,)), ((), ())),
                preferred_element_type=jnp.float32,
            )
        ),
        axis=1,
    )
    loss_u = jnp.mean(-jnp.log(pos_u / (neg_u + 1e-8) + 1e-8))

    scores = jnp.sum(zu3 * zi3, axis=2)
    sm = scores - jnp.max(scores, axis=1, keepdims=True)
    es = jnp.exp(sm)
    probs = es / jnp.sum(es, axis=1, keepdims=True)
    pm = jnp.max(probs, axis=1, keepdims=True)
    lse = pm + jnp.log(jnp.sum(jnp.exp(probs - pm), axis=1, keepdims=True))
    logp = probs - lse

    labv = lab[...]
    lm = jnp.max(labv, axis=1, keepdims=True)
    idxs = lax.broadcasted_iota(jnp.int32, (B, K), 1)
    cand = jnp.where(labv >= lm, idxs, K)
    tgt = jnp.min(cand, axis=1, keepdims=True)
    onehot = (idxs == tgt).astype(jnp.float32)
    rec = -jnp.mean(jnp.sum(logp * onehot, axis=1))

    reg = (jnp.sum(zu_b ** 2) + jnp.sum(zi ** 2)) * 0.5
    embl = reg * (L2_REG / B)

    scores_ref[...] = scores
    rec_ref[...] = jnp.full((1, 1), rec)
    embl_ref[...] = jnp.full((1, 1), embl)
    tot_ref[...] = jnp.full((1, 1), rec + embl + 0.5 * loss_u)


_BLK = 512
_NBLK = BF // _BLK


def _tc_body(
    emb_ref, svd_ref, uT, vT, utab, itab, lab,
    scores_ref, rec_ref, embl_ref, tot_ref, gnn_scr, hyp_scr,
):
    i = pl.program_id(0)

    @pl.when(i == 0)
    def _prep():
        _tc1_prep(
            emb_ref, svd_ref, uT, vT, utab, itab, lab,
            scores_ref, rec_ref, embl_ref, tot_ref, gnn_scr, hyp_scr,
        )

    @pl.when(i > 0)
    def _neg_i_block():
        g = gnn_scr[pl.ds((i - 1) * _BLK, _BLK), :]
        hr = hyp_scr[pl.ds((i - 1) * _BLK, _BLK), :]
        pos = jnp.exp(jnp.sum(g * hr, axis=1))
        neg = jnp.sum(
            jnp.exp(
                lax.dot_general(
                    g, hyp_scr[...], (((1,), (1,)), ((), ())),
                    preferred_element_type=jnp.float32,
                )
            ),
            axis=1,
        )
        s = jnp.sum(-jnp.log(pos / (neg + 1e-8) + 1e-8))
        tot_ref[...] = tot_ref[...] + jnp.full((1, 1), 0.5 * s / BF)


def kernel(user_table, item_table, u_svd, v_svd, users, items, label, ui_row, ui_col):
    users = users.astype(jnp.int32)
    items_flat = items.reshape(-1).astype(jnp.int32)
    ui_row = ui_row.astype(jnp.int32)
    ui_col = ui_col.astype(jnp.int32)

    u_svd_p = jnp.pad(u_svd, ((0, 0), (0, RPAD - RANK)))
    v_svd_p = jnp.pad(v_svd, ((0, 0), (0, RPAD - RANK)))

    row_idx = jnp.pad(ui_row, (0, NE_PAD - NE), constant_values=NU).reshape(
        NE_PAD // CHUNK, CR, SUB
    )
    col_idx = jnp.pad(ui_col, (0, NE_PAD - NE), constant_values=NU).reshape(
        NE_PAD // CHUNK, CR, SUB
    )

    users_rep = jnp.repeat(users, K)
    bidx0 = jnp.concatenate([users, users_rep])
    bidx1 = jnp.concatenate([items_flat, items_flat[:B]])
    bidx = jnp.stack([bidx0, bidx1]).reshape(NCORE, NSUB, GROWS, SUB)

    zrows = jnp.zeros((ZPT, D), jnp.float32)

    emb_b, svd_b = _get_sc_kernel()(
        item_table, user_table, row_idx, col_idx, bidx, u_svd_p, v_svd_p, zrows
    )

    def full(shape):
        return pl.BlockSpec(shape, lambda i, _n=len(shape): (0,) * _n)

    scores, rec, embl, tot = pl.pallas_call(
        _tc_body,
        grid=(_NBLK + 1,),
        in_specs=[
            full((NCORE, GB, D)), full((NCORE, GB, RPAD)),
            full((RPAD, NU)), full((RPAD, NI)),
            full((NU, D)), full((NI, D)), full((B, K)),
        ],
        out_specs=[full((B, K)), full((1, 1)), full((1, 1)), full((1, 1))],
        out_shape=(
            jax.ShapeDtypeStruct((B, K), jnp.float32),
            jax.ShapeDtypeStruct((1, 1), jnp.float32),
            jax.ShapeDtypeStruct((1, 1), jnp.float32),
            jax.ShapeDtypeStruct((1, 1), jnp.float32),
        ),
        scratch_shapes=[
            pltpu.VMEM((BF, D), jnp.float32),
            pltpu.VMEM((BF, D), jnp.float32),
        ],
    )(emb_b, svd_b, u_svd_p.T, v_svd_p.T, user_table, item_table, label)

    return (tot[0, 0], scores, rec[0, 0], embl[0, 0])
